# Initial kernel scaffold; baseline (speedup 1.0000x reference)
import jax
import jax.numpy as jnp
from jax.experimental import pallas as pl


def kernel(loop_idx, embedding_table):
    raise NotImplementedError("write your pallas kernel here")


# speedup vs baseline: 8.0206x; 8.0206x over previous
import functools
import math

import jax
import jax.numpy as jnp
import numpy as np
from jax import lax
from jax.experimental import pallas as pl
from jax.experimental.pallas import tpu as pltpu
from jax.experimental.pallas import tpu_sc as plsc

MAX_LOOPS = 100000
HIDDEN_DIM = 64


def _make_pe_np(max_loops: int, hidden_dim: int) -> np.ndarray:
    position = np.arange(0, max_loops, dtype=np.float32)[:, None]
    div_term = np.exp(
        np.arange(0, hidden_dim, 2, dtype=np.float32)
        * (-math.log(10000.0) / hidden_dim)
    )
    pe = np.zeros((max_loops, hidden_dim), dtype=np.float32)
    pe[:, 0::2] = np.sin(position * div_term)
    pe[:, 1::2] = np.cos(position * div_term)
    return pe


_PE = _make_pe_np(MAX_LOOPS, HIDDEN_DIM)

_A_ROWS = 50000
_A_BLK = 5000


def _add_body(t_ref, p_ref, o_ref):
    o_ref[...] = t_ref[...] + p_ref[...]


def _fuse_table(table):
    t2 = table.reshape(_A_ROWS, 128)
    p2 = _PE.reshape(_A_ROWS, 128)
    return pl.pallas_call(
        _add_body,
        grid=(_A_ROWS // _A_BLK,),
        in_specs=[
            pl.BlockSpec((_A_BLK, 128), lambda i: (i, 0)),
            pl.BlockSpec((_A_BLK, 128), lambda i: (i, 0)),
        ],
        out_specs=pl.BlockSpec((_A_BLK, 128), lambda i: (i, 0)),
        out_shape=jax.ShapeDtypeStruct((_A_ROWS, 128), jnp.float32),
    )(t2, p2)


_B = 4096 * 200
_D = HIDDEN_DIM
_NW = 32
_BPW = _B // _NW
_C = 1024
_NCH = _BPW // _C


def _gather(fused, idx):
    mesh = plsc.VectorSubcoreMesh(core_axis_name="c", subcore_axis_name="s")

    @functools.partial(
        pl.kernel,
        out_type=jax.ShapeDtypeStruct((_B, _D), jnp.float32),
        mesh=mesh,
        scratch_types=[
            pltpu.VMEM((_C,), jnp.int32),
            pltpu.VMEM((_C, _D), jnp.float32),
            pltpu.SemaphoreType.DMA,
        ],
        compiler_params=pltpu.CompilerParams(use_tc_tiling_on_sc=False),
    )
    def k(fused_hbm, idx_hbm, out_hbm, idx_v, rows_v, sem):
        wid = lax.axis_index("s") * 2 + lax.axis_index("c")
        base = wid * _BPW

        def body(g, carry):
            off = base + g * _C
            pltpu.sync_copy(idx_hbm.at[pl.ds(off, _C)], idx_v)
            pltpu.async_copy(fused_hbm.at[idx_v], rows_v, sem).wait()
            pltpu.sync_copy(rows_v, out_hbm.at[pl.ds(off, _C)])
            return carry

        lax.fori_loop(0, _NCH, body, 0)

    return k(fused, idx)


def kernel(loop_idx, embedding_table):
    idx = jnp.minimum(loop_idx, MAX_LOOPS - 1).reshape(-1)
    fused2 = _fuse_table(embedding_table)
    fused = fused2.reshape(MAX_LOOPS, _D)
    out = _gather(fused, idx)
    return out.reshape(4096, 200, _D)
